# baseline (speedup 1.0000x reference)
import functools

import jax
import jax.numpy as jnp
from jax import lax
from jax.experimental import pallas as pl
from jax.experimental.pallas import tpu as pltpu
from jax.experimental.pallas import tpu_sc as plsc

V, E, NCLS, B, S = 1000, 64, 10, 4096, 200
NC, NS, L = 2, 16, 16
NW = NC * NS
BPW = B // NW
HS = S // 2
WC = 16
NFULL = S // (2 * WC)
NREM = S - NFULL * 2 * WC
EW = L
OW = E // 2
CG = 32
RPB = 128 // CG

_mesh = plsc.VectorSubcoreMesh(
    core_axis_name="c", subcore_axis_name="s", num_cores=2)


def _bcast_lane(vec, j):
    return lax.gather(
        vec,
        jnp.full((L, 1), j, jnp.int32),
        lax.GatherDimensionNumbers(
            offset_dims=(), collapsed_slice_dims=(0,), start_index_map=(0,)),
        (1,),
        mode=lax.GatherScatterMode.PROMISE_IN_BOUNDS,
    )


@functools.partial(
    pl.kernel,
    out_type=jax.ShapeDtypeStruct((B * OW,), jnp.int32),
    mesh=_mesh,
    scratch_types=[
        pltpu.VMEM((BPW * HS + WC - NREM // 2,), jnp.int32),
        pltpu.VMEM((V * EW,), jnp.int32),
        pltpu.VMEM((V * EW,), jnp.int32),
        pltpu.VMEM((BPW * OW,), jnp.int32),
    ],
    compiler_params=pltpu.CompilerParams(needs_layout_passes=False),
)
def _pool_sc(idx_hbm, ta_hbm, tb_hbm, out_hbm, idx_v, ta_v, tb_v, out_v):
    wid = lax.axis_index("s") * NC + lax.axis_index("c")
    base = wid * BPW
    pltpu.sync_copy(ta_hbm, ta_v)
    pltpu.sync_copy(tb_hbm, tb_v)
    pltpu.sync_copy(idx_hbm.at[pl.ds(base * HS, BPW * HS)],
                    idx_v.at[pl.ds(0, BPW * HS)])
    lanes = lax.iota(jnp.int32, L)
    ninf = jnp.full((2 * L,), -jnp.inf, jnp.bfloat16)

    def pair_max(idxw, w, accs):
        a0, b0, a1, b1 = accs
        bc = _bcast_lane(idxw, w)
        ae = (bc & 0xFFFF) + lanes
        ao = lax.shift_right_logical(bc, 16) + lanes
        a0 = jnp.maximum(a0, plsc.bitcast(
            plsc.load_gather(ta_v, [ae]), jnp.bfloat16))
        b0 = jnp.maximum(b0, plsc.bitcast(
            plsc.load_gather(tb_v, [ae]), jnp.bfloat16))
        a1 = jnp.maximum(a1, plsc.bitcast(
            plsc.load_gather(ta_v, [ao]), jnp.bfloat16))
        b1 = jnp.maximum(b1, plsc.bitcast(
            plsc.load_gather(tb_v, [ao]), jnp.bfloat16))
        return (a0, b0, a1, b1)

    def row_body(row, _):
        def chunk_body(c, accs):
            idxw = idx_v[pl.ds(row * HS + c * WC, WC)]
            for w in range(WC):
                accs = pair_max(idxw, w, accs)
            return accs

        accs = lax.fori_loop(
            0, NFULL, chunk_body, (ninf, ninf, ninf, ninf))
        idxw = idx_v[pl.ds(row * HS + NFULL * WC, WC)]
        for w in range(NREM // 2):
            accs = pair_max(idxw, w, accs)
        a0, b0, a1, b1 = accs
        aa, ab = jnp.maximum(a0, a1), jnp.maximum(b0, b1)
        out_v[pl.ds(row * OW, L)] = plsc.bitcast(aa, jnp.int32)
        out_v[pl.ds(row * OW + L, L)] = plsc.bitcast(ab, jnp.int32)
        return 0

    lax.fori_loop(0, BPW, row_body, 0)
    pltpu.sync_copy(out_v, out_hbm.at[pl.ds(base * OW, BPW * OW)])


BLKR = 256


def _dense_tc(x_ref, we_ref, wo_ref, s_ref, b_ref, o_ref):
    w = x_ref[...]
    xe = lax.bitcast_convert_type(w << 16, jnp.float32)
    xo = lax.bitcast_convert_type(
        w & jnp.int32(-65536), jnp.float32)
    logits = (jnp.dot(xe, we_ref[...], preferred_element_type=jnp.float32)
              + jnp.dot(xo, wo_ref[...], preferred_element_type=jnp.float32)
              + b_ref[...])
    e = jnp.exp(logits)
    s = jnp.dot(e, s_ref[...], preferred_element_type=jnp.float32,
                precision=lax.Precision.HIGHEST)
    o_ref[...] = e / s


_dense_call = pl.pallas_call(
    _dense_tc,
    grid=(B * OW // 128 // BLKR,),
    in_specs=[
        pl.BlockSpec((BLKR, 128), lambda i: (i, 0)),
        pl.BlockSpec((128, 128), lambda i: (0, 0)),
        pl.BlockSpec((128, 128), lambda i: (0, 0)),
        pl.BlockSpec((128, 128), lambda i: (0, 0)),
        pl.BlockSpec((1, 128), lambda i: (0, 0)),
    ],
    out_specs=pl.BlockSpec((BLKR, 128), lambda i: (i, 0)),
    out_shape=jax.ShapeDtypeStruct((B * OW // 128, 128), jnp.float32),
)


def kernel(indices, table, W, b):
    tp = lax.bitcast_convert_type(
        table.astype(jnp.bfloat16).reshape(V, 2, EW, 2), jnp.int32)
    ta = tp[:, 0].reshape(-1)
    tb = tp[:, 1].reshape(-1)
    idx_p = lax.bitcast_convert_type(
        (indices * EW).astype(jnp.uint16).reshape(B * S // 2, 2), jnp.int32)
    pooled = _pool_sc(idx_p, ta, tb)
    w_e = jnp.kron(jnp.eye(RPB, dtype=jnp.float32),
                   jnp.pad(W[0::2], ((0, 0), (0, CG - NCLS))))
    w_o = jnp.kron(jnp.eye(RPB, dtype=jnp.float32),
                   jnp.pad(W[1::2], ((0, 0), (0, CG - NCLS))))
    s_bd = jnp.kron(jnp.eye(RPB, dtype=jnp.float32),
                    jnp.ones((CG, CG), jnp.float32))
    b_bd = jnp.tile(jnp.concatenate(
        [b, jnp.full((CG - NCLS,), -1e30, jnp.float32)]), RPB).reshape(1, 128)
    probs = _dense_call(pooled.reshape(B * OW // 128, 128),
                        w_e, w_o, s_bd, b_bd)
    return probs.reshape(B // RPB, RPB, CG)[:, :, :NCLS].reshape(B, NCLS)


# speedup vs baseline: 2.5386x; 2.5386x over previous
import functools

import jax
import jax.numpy as jnp
from jax import lax
from jax.experimental import pallas as pl
from jax.experimental.pallas import tpu as pltpu
from jax.experimental.pallas import tpu_sc as plsc

V, E, NCLS, B, S = 1000, 64, 10, 4096, 200
NC, NS, L = 2, 16, 16
NW = NC * NS
BPW = B // NW
HS = S // 2
WC = 16
NFULL = S // (2 * WC)
NREM = S - NFULL * 2 * WC
EW = L
OW = E // 2
CG = 32
RPB = 128 // CG

_mesh = plsc.VectorSubcoreMesh(
    core_axis_name="c", subcore_axis_name="s", num_cores=2)


def _bcast_lane(vec, j):
    return lax.gather(
        vec,
        jnp.full((L, 1), j, jnp.int32),
        lax.GatherDimensionNumbers(
            offset_dims=(), collapsed_slice_dims=(0,), start_index_map=(0,)),
        (1,),
        mode=lax.GatherScatterMode.PROMISE_IN_BOUNDS,
    )


@functools.partial(
    pl.kernel,
    out_type=jax.ShapeDtypeStruct((B * OW,), jnp.int32),
    mesh=_mesh,
    scratch_types=[
        pltpu.VMEM((BPW * HS + WC - NREM // 2,), jnp.int32),
        pltpu.VMEM((V * EW,), jnp.int32),
        pltpu.VMEM((V * EW,), jnp.int32),
        pltpu.VMEM((BPW * OW,), jnp.int32),
    ],
    compiler_params=pltpu.CompilerParams(needs_layout_passes=False),
)
def _pool_sc(idx_hbm, ta_hbm, tb_hbm, out_hbm, idx_v, ta_v, tb_v, out_v):
    wid = lax.axis_index("s") * NC + lax.axis_index("c")
    base = wid * BPW
    pltpu.sync_copy(ta_hbm, ta_v)
    pltpu.sync_copy(tb_hbm, tb_v)
    pltpu.sync_copy(idx_hbm.at[pl.ds(base * HS, BPW * HS)],
                    idx_v.at[pl.ds(0, BPW * HS)])
    lanes = lax.iota(jnp.int32, L)
    ninf = jnp.full((2 * L,), -jnp.inf, jnp.bfloat16)

    def pair_max(idxw, w, accs):
        a0, b0, a1, b1 = accs
        bc = _bcast_lane(idxw, w)
        ae = (bc & 0xFFFF) + lanes
        ao = lax.shift_right_logical(bc, 16) + lanes
        a0 = jnp.maximum(a0, plsc.bitcast(
            plsc.load_gather(ta_v, [ae]), jnp.bfloat16))
        b0 = jnp.maximum(b0, plsc.bitcast(
            plsc.load_gather(tb_v, [ae]), jnp.bfloat16))
        a1 = jnp.maximum(a1, plsc.bitcast(
            plsc.load_gather(ta_v, [ao]), jnp.bfloat16))
        b1 = jnp.maximum(b1, plsc.bitcast(
            plsc.load_gather(tb_v, [ao]), jnp.bfloat16))
        return (a0, b0, a1, b1)

    def row_body(row, _):
        def chunk_body(c, accs):
            idxw = idx_v[pl.ds(row * HS + c * WC, WC)]
            for w in range(WC):
                accs = pair_max(idxw, w, accs)
            return accs

        accs = lax.fori_loop(
            0, NFULL, chunk_body, (ninf, ninf, ninf, ninf))
        idxw = idx_v[pl.ds(row * HS + NFULL * WC, WC)]
        for w in range(NREM // 2):
            accs = pair_max(idxw, w, accs)
        a0, b0, a1, b1 = accs
        aa, ab = jnp.maximum(a0, a1), jnp.maximum(b0, b1)
        out_v[pl.ds(row * OW, L)] = plsc.bitcast(aa, jnp.int32)
        out_v[pl.ds(row * OW + L, L)] = plsc.bitcast(ab, jnp.int32)
        return 0

    lax.fori_loop(0, BPW, row_body, 0)
    pltpu.sync_copy(out_v, out_hbm.at[pl.ds(base * OW, BPW * OW)])


BLKR = 256


def _dense_tc(x_ref, we_ref, wo_ref, s_ref, b_ref, o_ref):
    w = x_ref[...]
    xe = lax.bitcast_convert_type(w << 16, jnp.float32)
    xo = lax.bitcast_convert_type(
        w & jnp.int32(-65536), jnp.float32)
    logits = (jnp.dot(xe, we_ref[...], preferred_element_type=jnp.float32)
              + jnp.dot(xo, wo_ref[...], preferred_element_type=jnp.float32)
              + b_ref[...])
    e = jnp.exp(logits)
    s = jnp.dot(e, s_ref[...], preferred_element_type=jnp.float32,
                precision=lax.Precision.HIGHEST)
    o_ref[...] = e / s


_dense_call = pl.pallas_call(
    _dense_tc,
    grid=(B * OW // 128 // BLKR,),
    in_specs=[
        pl.BlockSpec((BLKR, 128), lambda i: (i, 0)),
        pl.BlockSpec((128, 128), lambda i: (0, 0)),
        pl.BlockSpec((128, 128), lambda i: (0, 0)),
        pl.BlockSpec((128, 128), lambda i: (0, 0)),
        pl.BlockSpec((1, 128), lambda i: (0, 0)),
    ],
    out_specs=pl.BlockSpec((BLKR, 128), lambda i: (i, 0)),
    out_shape=jax.ShapeDtypeStruct((B * OW // 128, 128), jnp.float32),
)


def kernel(indices, table, W, b):
    tp = lax.bitcast_convert_type(
        table.astype(jnp.bfloat16).reshape(V, 2, EW, 2), jnp.int32)
    ta = tp[:, 0].reshape(-1)
    tb = tp[:, 1].reshape(-1)
    idx_p = (indices[:, 0::2] * EW
             + indices[:, 1::2] * (EW * 65536)).reshape(-1)
    pooled = _pool_sc(idx_p, ta, tb)
    w_e = jnp.kron(jnp.eye(RPB, dtype=jnp.float32),
                   jnp.pad(W[0::2], ((0, 0), (0, CG - NCLS))))
    w_o = jnp.kron(jnp.eye(RPB, dtype=jnp.float32),
                   jnp.pad(W[1::2], ((0, 0), (0, CG - NCLS))))
    s_bd = jnp.kron(jnp.eye(RPB, dtype=jnp.float32),
                    jnp.ones((CG, CG), jnp.float32))
    b_bd = jnp.tile(jnp.concatenate(
        [b, jnp.full((CG - NCLS,), -1e30, jnp.float32)]), RPB).reshape(1, 128)
    probs = _dense_call(pooled.reshape(B * OW // 128, 128),
                        w_e, w_o, s_bd, b_bd)
    return probs.reshape(B // RPB, RPB, CG)[:, :, :NCLS].reshape(B, NCLS)


# speedup vs baseline: 2.8461x; 1.1211x over previous
import functools

import jax
import jax.numpy as jnp
from jax import lax
from jax.experimental import pallas as pl
from jax.experimental.pallas import tpu as pltpu
from jax.experimental.pallas import tpu_sc as plsc

V, E, NCLS, B, S = 1000, 64, 10, 4096, 200
NC, NS, L = 2, 16, 16
NW = NC * NS
BPW = B // NW
NFULL = S // L
NREM = S - NFULL * L
EW = L
OW = E // 2
CG = 32
RPB = 128 // CG

_mesh = plsc.VectorSubcoreMesh(
    core_axis_name="c", subcore_axis_name="s", num_cores=2)


def _bcast_lane(vec, j):
    return lax.gather(
        vec,
        jnp.full((L, 1), j, jnp.int32),
        lax.GatherDimensionNumbers(
            offset_dims=(), collapsed_slice_dims=(0,), start_index_map=(0,)),
        (1,),
        mode=lax.GatherScatterMode.PROMISE_IN_BOUNDS,
    )


@functools.partial(
    pl.kernel,
    out_type=jax.ShapeDtypeStruct((B * OW,), jnp.int32),
    mesh=_mesh,
    scratch_types=[
        pltpu.VMEM((BPW * S + L - NREM,), jnp.int32),
        pltpu.VMEM((V * EW,), jnp.int32),
        pltpu.VMEM((V * EW,), jnp.int32),
        pltpu.VMEM((BPW * OW,), jnp.int32),
    ],
    compiler_params=pltpu.CompilerParams(needs_layout_passes=False),
)
def _pool_sc(idx_hbm, ta_hbm, tb_hbm, out_hbm, idx_v, ta_v, tb_v, out_v):
    wid = lax.axis_index("s") * NC + lax.axis_index("c")
    base = wid * BPW
    pltpu.sync_copy(ta_hbm, ta_v)
    pltpu.sync_copy(tb_hbm, tb_v)
    pltpu.sync_copy(idx_hbm.at[pl.ds(base * S, BPW * S)],
                    idx_v.at[pl.ds(0, BPW * S)])
    lanes = lax.iota(jnp.int32, L)
    ninf = jnp.full((2 * L,), -jnp.inf, jnp.bfloat16)

    def gather_max(idxs, j, aa, ab):
        addr = _bcast_lane(idxs, j) + lanes
        wa = plsc.bitcast(plsc.load_gather(ta_v, [addr]), jnp.bfloat16)
        wb = plsc.bitcast(plsc.load_gather(tb_v, [addr]), jnp.bfloat16)
        return jnp.maximum(aa, wa), jnp.maximum(ab, wb)

    def row_body(row, _):
        def chunk_body(c, accs):
            idxs = idx_v[pl.ds(row * S + c * L, L)] * EW
            a0, b0, a1, b1 = accs
            for j in range(0, L, 2):
                a0, b0 = gather_max(idxs, j, a0, b0)
                a1, b1 = gather_max(idxs, j + 1, a1, b1)
            return (a0, b0, a1, b1)

        a0, b0, a1, b1 = lax.fori_loop(
            0, NFULL, chunk_body, (ninf, ninf, ninf, ninf))
        idxs = idx_v[pl.ds(row * S + NFULL * L, L)] * EW
        for j in range(0, NREM, 2):
            a0, b0 = gather_max(idxs, j, a0, b0)
            a1, b1 = gather_max(idxs, j + 1, a1, b1)
        aa, ab = jnp.maximum(a0, a1), jnp.maximum(b0, b1)
        out_v[pl.ds(row * OW, L)] = plsc.bitcast(aa, jnp.int32)
        out_v[pl.ds(row * OW + L, L)] = plsc.bitcast(ab, jnp.int32)
        return 0

    lax.fori_loop(0, BPW, row_body, 0)
    pltpu.sync_copy(out_v, out_hbm.at[pl.ds(base * OW, BPW * OW)])


BLKR = 256


def _dense_tc(x_ref, we_ref, wo_ref, s_ref, b_ref, o_ref):
    w = x_ref[...]
    xe = lax.bitcast_convert_type(w << 16, jnp.float32)
    xo = lax.bitcast_convert_type(
        w & jnp.int32(-65536), jnp.float32)
    logits = (jnp.dot(xe, we_ref[...], preferred_element_type=jnp.float32)
              + jnp.dot(xo, wo_ref[...], preferred_element_type=jnp.float32)
              + b_ref[...])
    e = jnp.exp(logits)
    s = jnp.dot(e, s_ref[...], preferred_element_type=jnp.float32,
                precision=lax.Precision.HIGHEST)
    o_ref[...] = e / s


_dense_call = pl.pallas_call(
    _dense_tc,
    grid=(B * OW // 128 // BLKR,),
    in_specs=[
        pl.BlockSpec((BLKR, 128), lambda i: (i, 0)),
        pl.BlockSpec((128, 128), lambda i: (0, 0)),
        pl.BlockSpec((128, 128), lambda i: (0, 0)),
        pl.BlockSpec((128, 128), lambda i: (0, 0)),
        pl.BlockSpec((1, 128), lambda i: (0, 0)),
    ],
    out_specs=pl.BlockSpec((BLKR, 128), lambda i: (i, 0)),
    out_shape=jax.ShapeDtypeStruct((B * OW // 128, 128), jnp.float32),
)


def kernel(indices, table, W, b):
    tp = lax.bitcast_convert_type(
        table.astype(jnp.bfloat16).reshape(V, 2, EW, 2), jnp.int32)
    ta = tp[:, 0].reshape(-1)
    tb = tp[:, 1].reshape(-1)
    pooled = _pool_sc(indices.reshape(-1), ta, tb)
    w_e = jnp.kron(jnp.eye(RPB, dtype=jnp.float32),
                   jnp.pad(W[0::2], ((0, 0), (0, CG - NCLS))))
    w_o = jnp.kron(jnp.eye(RPB, dtype=jnp.float32),
                   jnp.pad(W[1::2], ((0, 0), (0, CG - NCLS))))
    s_bd = jnp.kron(jnp.eye(RPB, dtype=jnp.float32),
                    jnp.ones((CG, CG), jnp.float32))
    b_bd = jnp.tile(jnp.concatenate(
        [b, jnp.full((CG - NCLS,), -1e30, jnp.float32)]), RPB).reshape(1, 128)
    probs = _dense_call(pooled.reshape(B * OW // 128, 128),
                        w_e, w_o, s_bd, b_bd)
    return probs.reshape(B // RPB, RPB, CG)[:, :, :NCLS].reshape(B, NCLS)
